# baseline (speedup 1.0000x reference)
import jax
import jax.numpy as jnp
from jax.experimental import pallas as pl
from jax.experimental.pallas import tpu as pltpu

_EPS = 1e-6
_GRID = 128


def _shl(x, k):
    z = jnp.zeros(x.shape[:-1] + (k,), x.dtype)
    return jnp.concatenate([x[..., k:], z], axis=-1)


def _shr(x, k):
    z = jnp.zeros(x.shape[:-1] + (k,), x.dtype)
    return jnp.concatenate([z, x[..., :-k]], axis=-1)


def _block_loss(p, t):
    c = jax.lax.broadcasted_iota(jnp.int32, (1, 1, 1, 30), 3)
    box_lane = c < 10
    wh_lane = (c == 2) | (c == 3) | (c == 7) | (c == 8)
    conf_lane = (c == 4) | (c == 9)
    xy_lane = (c == 0) | (c == 1) | (c == 5) | (c == 6)
    coef = jnp.where(wh_lane | xy_lane, 5.0, 1.0).astype(jnp.float32)

    t_rep = jnp.where((c >= 5) & box_lane, _shr(t, 5), t)

    pw = _shl(0.5 * p, 2)
    tw = _shl(0.5 * t_rep, 2)
    ov = jnp.maximum(
        jnp.minimum(p + pw, t_rep + tw) - jnp.maximum(p - pw, t_rep - tw),
        0.0)
    inter = ov * _shl(ov, 1)
    area = p * _shl(p, 1) + t_rep * _shl(t_rep, 1)
    union = _shl(area, 2) - inter
    iou = inter / (union + _EPS)
    m = jnp.where(iou > 0, iou, 0.0)

    sel = m[..., 5:6] > m[..., 0:1]
    sel_f = jnp.where(sel, 1.0, 0.0)
    selw = jnp.where(c < 5, 1.0 - sel_f, sel_f)

    obj = jnp.where(t[..., 4:5] > 0, 1.0, 0.0)

    a = p - t_rep
    a = a * a
    w_ = jnp.sqrt(jnp.maximum(p, _EPS)) - jnp.sqrt(jnp.maximum(t_rep, _EPS))
    w_ = w_ * w_
    base = jnp.where(wh_lane, w_, a)

    wsel = jnp.where(box_lane, selw, 1.0)
    contrib = base * (obj * wsel * coef)

    psq = p * p
    noobj = 0.5 * psq * (1.0 - obj * selw)
    contrib = contrib + jnp.where(conf_lane, noobj, 0.0)
    return jnp.sum(contrib, axis=(0, 1, 2, 3), keepdims=True)


def _loss_kernel(p_ref, t_ref, o_ref):
    o_ref[...] = _block_loss(p_ref[...], t_ref[...]).reshape(1, 1, 1)


def kernel(predictions, targets):
    n, s1, s2, ch = predictions.shape
    bn = n // _GRID
    partials = pl.pallas_call(
        _loss_kernel,
        grid=(_GRID,),
        in_specs=[
            pl.BlockSpec((bn, s1, s2, ch), lambda i: (i, 0, 0, 0)),
            pl.BlockSpec((bn, s1, s2, ch), lambda i: (i, 0, 0, 0)),
        ],
        out_specs=pl.BlockSpec((1, 1, 1), lambda i: (i, 0, 0)),
        out_shape=jax.ShapeDtypeStruct((_GRID, 1, 1), jnp.float32),
        compiler_params=pltpu.CompilerParams(
            dimension_semantics=("parallel",)),
    )(predictions, targets)
    return jnp.sum(partials) / n
